# baseline (speedup 1.0000x reference)
import functools

import jax
import jax.numpy as jnp
from jax import lax
from jax.experimental import pallas as pl
from jax.experimental.pallas import tpu as pltpu
from jax.experimental.pallas import tpu_sc as plsc

M_POINTS = 1024
GRASP_THR = 0.1
N_REAL = 50000
NPAD = 51200
SHARD = NPAD // 8
NB = 6400
C_DIM = 256
OUTW = C_DIM + 4
KMIN = -(2 ** 31)


def _head_body(f_ref, w_ref, b_ref, key_ref, grasp_ref):
    ni = pl.program_id(1)
    f = f_ref[0]
    w = w_ref[...]
    h = jax.lax.dot_general(
        w, f, (((1,), (0,)), ((), ())), preferred_element_type=jnp.float32
    )
    h = h + b_ref[...]
    obj = h[1:2] > h[0:1]
    g = h[2:3]
    keep = obj & (g > GRASP_THR)
    masked = jnp.where(keep, g, jnp.float32(-1e9))
    bits = jax.lax.bitcast_convert_type(masked, jnp.int32)
    key = jnp.where(bits >= 0, bits, bits ^ jnp.int32(0x7FFFFFFF))
    pos = jax.lax.broadcasted_iota(jnp.int32, (1, NB), 1) + ni * NB
    key = jnp.where(pos < N_REAL, key, jnp.int32(KMIN))
    key_ref[0] = key
    grasp_ref[0] = g


def _run_head(features, W, b):
    B, C, N = features.shape
    return pl.pallas_call(
        _head_body,
        grid=(B, NPAD // NB),
        in_specs=[
            pl.BlockSpec((1, C, NB), lambda bi, ni: (bi, 0, ni)),
            pl.BlockSpec((3, C), lambda bi, ni: (0, 0)),
            pl.BlockSpec((3, 1), lambda bi, ni: (0, 0)),
        ],
        out_specs=[
            pl.BlockSpec((1, 1, NB), lambda bi, ni: (bi, 0, ni)),
            pl.BlockSpec((1, 1, NB), lambda bi, ni: (bi, 0, ni)),
        ],
        out_shape=[
            jax.ShapeDtypeStruct((B, 1, NPAD), jnp.int32),
            jax.ShapeDtypeStruct((B, 1, NPAD), jnp.float32),
        ],
    )(features, W, b.reshape(3, 1))


_LANES = 16
_NVREG = SHARD // _LANES
_DUMP = 4 * M_POINTS


def _lane_iota():
    return jax.lax.broadcasted_iota(jnp.int32, (_LANES,), 0)


def _scal(v):
    return jax.lax.reduce_max(v, (0,))


def _sum_scal(v):
    return jax.lax.reduce_sum(v, (0,))


def _at_lane(v, pos):
    return _sum_scal(jnp.where(_lane_iota() == pos, v, 0))


def _select_body(keys_hbm, outk_hbm, outi_hbm, keys_v, hist_v, stage_v,
                 h8_v, cnt_v, c8_v, gtk_v, gti_v, eqi_v, posb_v, valb_v,
                 sh_hist, sh_cnt, sem):
    c = lax.axis_index("c")
    s = lax.axis_index("s")
    bloc = s // 8
    t = s % 8
    batch = c * 2 + bloc
    base = batch * NPAD + t * SHARD

    pltpu.sync_copy(keys_hbm.at[pl.ds(base, SHARD)], keys_v)

    li = _lane_iota()
    ones = jnp.ones((_LANES,), jnp.int32)
    zeros16 = jnp.zeros((_LANES,), jnp.int32)
    flip = jnp.full((_LANES,), jnp.uint32(0x80000000))

    def load_uk(i):
        k = keys_v[pl.ds(i * _LANES, _LANES)]
        return plsc.bitcast(k, jnp.uint32) ^ flip

    prefix = jnp.uint32(0)
    need = jnp.int32(M_POINTS)
    total_gt = jnp.int32(0)

    for r in range(4):
        shift = jnp.uint32(24 - 8 * r)

        def clr(i, _):
            hist_v[pl.ds(i * _LANES, _LANES)] = zeros16
            return 0
        lax.fori_loop(0, 4096 // _LANES, clr, 0)

        pfx = prefix

        def hbody(i, _):
            uk = load_uk(i)
            dig = plsc.bitcast((uk >> shift) & jnp.uint32(0xFF), jnp.int32)
            if r == 0:
                m = jnp.ones((_LANES,), jnp.bool_)
            else:
                m = (uk >> jnp.uint32(32 - 8 * r)) == pfx
            plsc.addupdate_scatter(hist_v, [li * 256 + dig], ones, mask=m)
            return 0
        lax.fori_loop(0, _NVREG, hbody, 0)

        def red(dv, _):
            acc = zeros16
            for l in range(16):
                acc = acc + hist_v[pl.ds(l * 256 + dv * _LANES, _LANES)]
            stage_v[pl.ds(dv * _LANES, _LANES)] = acc
            return 0
        lax.fori_loop(0, 16, red, 0)

        pltpu.sync_copy(stage_v, sh_hist.at[r, bloc, t])
        plsc.subcore_barrier()
        pltpu.sync_copy(sh_hist.at[r, bloc], h8_v)

        def scan_dv(j, carry):
            run, found, beta, gt_at, sfx_at = carry
            dv = 15 - j
            tot = zeros16
            for tt in range(8):
                tot = tot + h8_v[tt, pl.ds(dv * _LANES, _LANES)]
            rev = lax.rev(tot, (0,))
            cum = plsc.cumsum(rev)
            cond = (run + cum) >= need
            pop = _scal(plsc.all_reduce_population_count(cond))
            has = pop > 0
            pos = plsc.all_reduce_ffs(cond)
            pos = jnp.where(has, _scal(pos), 0)
            cpos = _at_lane(cum, pos)
            rpos = _at_lane(rev, pos)
            hit = jnp.logical_and(has, jnp.logical_not(found))
            beta = jnp.where(hit, dv * 16 + (15 - pos), beta)
            sfx = run + cpos
            gt_at = jnp.where(hit, sfx - rpos, gt_at)
            sfx_at = jnp.where(hit, sfx, sfx_at)
            found = jnp.logical_or(found, has)
            run = run + _at_lane(cum, 15)
            return run, found, beta, gt_at, sfx_at

        init = (jnp.int32(0), jnp.bool_(False), jnp.int32(0), jnp.int32(0),
                jnp.int32(0))
        _, _, beta, gt_r, _ = lax.fori_loop(0, 16, scan_dv, init)

        prefix = (pfx << jnp.uint32(8)) | jnp.uint32(beta.astype(jnp.uint32))
        need = need - gt_r
        total_gt = total_gt + gt_r

    T_u = prefix
    need_eq = need

    def cbody(i, carry):
        off_gt, off_eq = carry
        uk = load_uk(i)
        kv = keys_v[pl.ds(i * _LANES, _LANES)]
        nv = t * SHARD + i * _LANES + li
        m_gt = uk > T_u
        m_eq = uk == T_u
        plsc.store_compressed(gtk_v.at[pl.ds(off_gt, _LANES)], kv, mask=m_gt)
        plsc.store_compressed(gti_v.at[pl.ds(off_gt, _LANES)], nv, mask=m_gt)

        @pl.when(off_eq < M_POINTS)
        def _():
            plsc.store_compressed(eqi_v.at[pl.ds(off_eq, _LANES)], nv,
                                  mask=m_eq)

        off_gt = off_gt + _scal(plsc.all_reduce_population_count(m_gt))
        off_eq = off_eq + _scal(plsc.all_reduce_population_count(m_eq))
        return off_gt, off_eq

    gt_cnt, eq_cnt = lax.fori_loop(0, _NVREG, cbody,
                                   (jnp.int32(0), jnp.int32(0)))

    cnt_v[...] = jnp.where(li == 0, gt_cnt,
                           jnp.where(li == 1, eq_cnt, jnp.int32(0)))
    pltpu.sync_copy(cnt_v, sh_cnt.at[bloc, t])
    plsc.subcore_barrier()
    pltpu.sync_copy(sh_cnt.at[bloc], c8_v)

    gt_base = jnp.int32(0)
    eq_base = jnp.int32(0)
    gt_tot = jnp.int32(0)
    for tt in range(8):
        row = c8_v[tt]
        gcnt = row[0]
        ecnt = row[1]
        before = jnp.int32(tt) < t
        gt_base = gt_base + jnp.where(before, gcnt, 0)
        eq_base = eq_base + jnp.where(before, ecnt, 0)
        gt_tot = gt_tot + gcnt

    wid = c * 16 + s
    obase = batch * M_POINTS

    def _scatter(dst_hbm):
        for j in range(8):
            pltpu.async_copy(valb_v.at[pl.ds(j * 128, 128)],
                             dst_hbm.at[posb_v.at[j]], sem)
        pltpu.make_async_copy(dst_hbm.at[pl.ds(0, M_POINTS)], valb_v,
                              sem).wait()

    for j in range(M_POINTS // _LANES):
        jj = j * _LANES + li
        valid = jj < gt_cnt
        pos = jnp.where(valid, obase + gt_base + jj, _DUMP + wid * 16 + li)
        posb_v[j // 8, pl.ds((j % 8) * _LANES, _LANES)] = pos
        valb_v[pl.ds(j * _LANES, _LANES)] = gtk_v[pl.ds(j * _LANES, _LANES)]
    _scatter(outk_hbm)
    for j in range(M_POINTS // _LANES):
        valb_v[pl.ds(j * _LANES, _LANES)] = gti_v[pl.ds(j * _LANES, _LANES)]
    _scatter(outi_hbm)

    t_signed = plsc.bitcast(
        jnp.zeros((_LANES,), jnp.uint32) + (T_u ^ jnp.uint32(0x80000000)),
        jnp.int32)
    for j in range(M_POINTS // _LANES):
        jj = j * _LANES + li
        erank = eq_base + jj
        valid = jnp.logical_and(jj < eq_cnt, erank < need_eq)
        pos = jnp.where(valid, obase + gt_tot + erank,
                        _DUMP + wid * 16 + li)
        posb_v[j // 8, pl.ds((j % 8) * _LANES, _LANES)] = pos
        valb_v[pl.ds(j * _LANES, _LANES)] = eqi_v[pl.ds(j * _LANES, _LANES)]
    _scatter(outi_hbm)
    for j in range(M_POINTS // _LANES):
        valb_v[pl.ds(j * _LANES, _LANES)] = t_signed
    _scatter(outk_hbm)


def _run_select(keys_flat):
    mesh = plsc.VectorSubcoreMesh(core_axis_name="c", subcore_axis_name="s")
    f = pl.kernel(
        _select_body,
        compiler_params=pltpu.CompilerParams(needs_layout_passes=False),
        out_type=[
            jax.ShapeDtypeStruct((4 * M_POINTS + 512,), jnp.int32),
            jax.ShapeDtypeStruct((4 * M_POINTS + 512,), jnp.int32),
        ],
        mesh=mesh,
        scratch_types=[
            pltpu.VMEM((SHARD,), jnp.int32),
            pltpu.VMEM((4096,), jnp.int32),
            pltpu.VMEM((256,), jnp.int32),
            pltpu.VMEM((8, 256), jnp.int32),
            pltpu.VMEM((16,), jnp.int32),
            pltpu.VMEM((8, 16), jnp.int32),
            pltpu.VMEM((M_POINTS + 16,), jnp.int32),
            pltpu.VMEM((M_POINTS + 16,), jnp.int32),
            pltpu.VMEM((M_POINTS + 16,), jnp.int32),
            pltpu.VMEM((8, 128), jnp.int32),
            pltpu.VMEM((M_POINTS,), jnp.int32),
            pltpu.VMEM_SHARED((4, 2, 8, 256), jnp.int32),
            pltpu.VMEM_SHARED((2, 8, 16), jnp.int32),
            pltpu.SemaphoreType.DMA,
        ],
    )
    return f(keys_flat)


def _rank_body(kj_ref, ki_ref, ci_ref, out_ref):
    kj = kj_ref[0]
    ki = ki_ref[0]
    gt = (kj > ki).astype(jnp.int32)
    ii = jax.lax.broadcasted_iota(jnp.int32, (M_POINTS, M_POINTS), 0)
    jj = jax.lax.broadcasted_iota(jnp.int32, (M_POINTS, M_POINTS), 1)
    eq = jnp.logical_and(kj == ki, jj < ii).astype(jnp.int32)
    rank = jnp.sum(gt + eq, axis=1, keepdims=True)
    onehot = (rank == jj).astype(jnp.float32)
    ci = ci_ref[0].astype(jnp.float32)
    sorted_f = jax.lax.dot_general(
        ci, onehot, (((1,), (0,)), ((), ())),
        preferred_element_type=jnp.float32)
    out_ref[0] = sorted_f.astype(jnp.int32)


def _run_rank(ck, ckT, ci):
    B = ck.shape[0]
    return pl.pallas_call(
        _rank_body,
        grid=(B,),
        in_specs=[
            pl.BlockSpec((1, 1, M_POINTS), lambda bi: (bi, 0, 0)),
            pl.BlockSpec((1, M_POINTS, 1), lambda bi: (bi, 0, 0)),
            pl.BlockSpec((1, 1, M_POINTS), lambda bi: (bi, 0, 0)),
        ],
        out_specs=pl.BlockSpec((1, 1, M_POINTS), lambda bi: (bi, 0, 0)),
        out_shape=jax.ShapeDtypeStruct((B, 1, M_POINTS), jnp.int32),
    )(ck, ckT, ci)


_PTS = 128
_DEBUG_DUMP = False


def _gather_body(pc_hbm, ft_hbm, gr_hbm, si_hbm, fo_hbm, xo_hbm,
                 sidx_v, fidx_v, orow_v, xbuf_v, pci_v, sem):
    c = lax.axis_index("c")
    s = lax.axis_index("s")
    wid = s * 2 + c
    batch = wid // 8
    m0 = (wid % 8) * _PTS

    li = _lane_iota()
    pltpu.sync_copy(si_hbm.at[pl.ds(batch * M_POINTS + m0, _PTS)], sidx_v)

    f_base = batch * C_DIM * N_REAL
    pc_base = batch * N_REAL * 3
    g_base = batch * NPAD

    def grow(p, _):
        nv = sidx_v[pl.ds((p // _LANES) * _LANES, _LANES)]
        n_p = _sum_scal(jnp.where(li == p % _LANES, nv, 0))
        base = f_base + n_p
        for v in range(C_DIM // _LANES):
            fidx_v[pl.ds(v * _LANES, _LANES)] = \
                base + (v * _LANES + li) * N_REAL
        pltpu.async_copy(ft_hbm.at[fidx_v],
                         orow_v.at[pl.ds(p * C_DIM, C_DIM)], sem).wait()
        return 0
    lax.fori_loop(0, _PTS, grow, 0)

    for e in range(3):
        for pv in range(_PTS // _LANES):
            nv = sidx_v[pl.ds(pv * _LANES, _LANES)]
            pci_v[pl.ds(pv * _LANES, _LANES)] = pc_base + nv * 3 + e
        pltpu.async_copy(pc_hbm.at[pci_v],
                         xbuf_v.at[pl.ds(e * _PTS, _PTS)], sem).wait()
    for pv in range(_PTS // _LANES):
        nv = sidx_v[pl.ds(pv * _LANES, _LANES)]
        pci_v[pl.ds(pv * _LANES, _LANES)] = g_base + nv
    pltpu.async_copy(gr_hbm.at[pci_v],
                     xbuf_v.at[pl.ds(3 * _PTS, _PTS)], sem).wait()

    pltpu.sync_copy(orow_v,
                    fo_hbm.at[pl.ds(wid * _PTS * C_DIM, _PTS * C_DIM)])
    pltpu.sync_copy(xbuf_v, xo_hbm.at[pl.ds(wid * 4 * _PTS, 4 * _PTS)])


def _run_gather(pc_flat, ft_flat, gr_flat, si_flat):
    mesh = plsc.VectorSubcoreMesh(core_axis_name="c", subcore_axis_name="s")
    f = pl.kernel(
        _gather_body,
        compiler_params=pltpu.CompilerParams(needs_layout_passes=False),
        out_type=[
            jax.ShapeDtypeStruct((4 * M_POINTS * C_DIM,), jnp.float32),
            jax.ShapeDtypeStruct((32 * 4 * _PTS,), jnp.float32),
        ],
        mesh=mesh,
        scratch_types=[
            pltpu.VMEM((_PTS,), jnp.int32),
            pltpu.VMEM((C_DIM,), jnp.int32),
            pltpu.VMEM((_PTS * C_DIM,), jnp.float32),
            pltpu.VMEM((4 * _PTS,), jnp.float32),
            pltpu.VMEM((_PTS,), jnp.int32),
            pltpu.SemaphoreType.DMA,
        ],
    )
    return f(pc_flat, ft_flat, gr_flat, si_flat)


def kernel(point_clouds, features, W, b):
    B, C, N = features.shape
    keys, grasp = _run_head(features, W, b)
    keys_flat = keys.reshape(B * NPAD)
    ck_flat, ci_flat = _run_select(keys_flat)
    ck = ck_flat[: B * M_POINTS].reshape(B, 1, M_POINTS)
    ckT = ck.reshape(B, M_POINTS, 1)
    ci = ci_flat[: B * M_POINTS].reshape(B, 1, M_POINTS)
    sidx = _run_rank(ck, ckT, ci)
    idx = sidx.reshape(B, M_POINTS)
    xyz = jnp.take_along_axis(point_clouds, idx[:, :, None], axis=1)
    feats = jnp.take_along_axis(
        jnp.transpose(features, (0, 2, 1)), idx[:, :, None], axis=1)
    scores = jnp.take_along_axis(
        grasp.reshape(B, NPAD), idx, axis=1)[:, :, None]
    return jnp.concatenate([xyz, feats, scores], axis=-1)


# speedup vs baseline: 1.0013x; 1.0013x over previous
import functools

import jax
import jax.numpy as jnp
from jax import lax
from jax.experimental import pallas as pl
from jax.experimental.pallas import tpu as pltpu
from jax.experimental.pallas import tpu_sc as plsc

M_POINTS = 1024
GRASP_THR = 0.1
N_REAL = 50000
NPAD = 51200
SHARD = NPAD // 8
NB = 6400
C_DIM = 256
OUTW = C_DIM + 4
KMIN = -(2 ** 31)


def _head_body(f_ref, w_ref, b_ref, key_ref, grasp_ref):
    ni = pl.program_id(1)
    f = f_ref[0]
    w = w_ref[...]
    h = jax.lax.dot_general(
        w, f, (((1,), (0,)), ((), ())), preferred_element_type=jnp.float32
    )
    h = h + b_ref[...]
    obj = h[1:2] > h[0:1]
    g = h[2:3]
    keep = obj & (g > GRASP_THR)
    masked = jnp.where(keep, g, jnp.float32(-1e9))
    bits = jax.lax.bitcast_convert_type(masked, jnp.int32)
    key = jnp.where(bits >= 0, bits, bits ^ jnp.int32(0x7FFFFFFF))
    pos = jax.lax.broadcasted_iota(jnp.int32, (1, NB), 1) + ni * NB
    key = jnp.where(pos < N_REAL, key, jnp.int32(KMIN))
    key_ref[0] = key
    grasp_ref[0] = g


def _run_head(features, W, b):
    B, C, N = features.shape
    return pl.pallas_call(
        _head_body,
        grid=(B, NPAD // NB),
        in_specs=[
            pl.BlockSpec((1, C, NB), lambda bi, ni: (bi, 0, ni)),
            pl.BlockSpec((3, C), lambda bi, ni: (0, 0)),
            pl.BlockSpec((3, 1), lambda bi, ni: (0, 0)),
        ],
        out_specs=[
            pl.BlockSpec((1, 1, NB), lambda bi, ni: (bi, 0, ni)),
            pl.BlockSpec((1, 1, NB), lambda bi, ni: (bi, 0, ni)),
        ],
        out_shape=[
            jax.ShapeDtypeStruct((B, 1, NPAD), jnp.int32),
            jax.ShapeDtypeStruct((B, 1, NPAD), jnp.float32),
        ],
    )(features, W, b.reshape(3, 1))


_LANES = 16
_NVREG = SHARD // _LANES
_DUMP = 4 * M_POINTS


def _lane_iota():
    return jax.lax.broadcasted_iota(jnp.int32, (_LANES,), 0)


def _scal(v):
    return jax.lax.reduce_max(v, (0,))


def _sum_scal(v):
    return jax.lax.reduce_sum(v, (0,))


def _at_lane(v, pos):
    return _sum_scal(jnp.where(_lane_iota() == pos, v, 0))


def _select_body(keys_hbm, outk_hbm, outi_hbm, keys_v, hist_v, stage_v,
                 h8_v, cnt_v, c8_v, gtk_v, gti_v, eqi_v, posb_v, valb_v,
                 sh_hist, sh_cnt, sem):
    c = lax.axis_index("c")
    s = lax.axis_index("s")
    bloc = s // 8
    t = s % 8
    batch = c * 2 + bloc
    base = batch * NPAD + t * SHARD

    pltpu.sync_copy(keys_hbm.at[pl.ds(base, SHARD)], keys_v)

    li = _lane_iota()
    ones = jnp.ones((_LANES,), jnp.int32)
    zeros16 = jnp.zeros((_LANES,), jnp.int32)
    flip = jnp.full((_LANES,), jnp.uint32(0x80000000))

    def load_uk(i):
        k = keys_v[pl.ds(pl.multiple_of(i * _LANES, _LANES), _LANES)]
        return plsc.bitcast(k, jnp.uint32) ^ flip

    prefix = jnp.uint32(0)
    need = jnp.int32(M_POINTS)
    total_gt = jnp.int32(0)

    for r in range(4):
        shift = jnp.uint32(24 - 8 * r)

        def clr(i, _):
            hist_v[pl.ds(pl.multiple_of(i * _LANES, _LANES), _LANES)] = zeros16
            return 0
        lax.fori_loop(0, 4096 // _LANES, clr, 0)

        pfx = prefix

        def hbody(i, _):
            uk = load_uk(i)
            dig = plsc.bitcast((uk >> shift) & jnp.uint32(0xFF), jnp.int32)
            if r == 0:
                m = jnp.ones((_LANES,), jnp.bool_)
            else:
                m = (uk >> jnp.uint32(32 - 8 * r)) == pfx
            plsc.addupdate_scatter(hist_v, [li * 256 + dig], ones, mask=m)
            return 0
        lax.fori_loop(0, _NVREG, hbody, 0)

        def red(dv, _):
            acc = zeros16
            for l in range(16):
                acc = acc + hist_v[pl.ds(pl.multiple_of(l * 256 + dv * _LANES, _LANES), _LANES)]
            stage_v[pl.ds(pl.multiple_of(dv * _LANES, _LANES), _LANES)] = acc
            return 0
        lax.fori_loop(0, 16, red, 0)

        pltpu.sync_copy(stage_v, sh_hist.at[r, bloc, t])
        plsc.subcore_barrier()
        pltpu.sync_copy(sh_hist.at[r, bloc], h8_v)

        def scan_dv(j, carry):
            run, found, beta, gt_at, sfx_at = carry
            dv = 15 - j
            tot = zeros16
            for tt in range(8):
                tot = tot + h8_v[tt, pl.ds(pl.multiple_of(dv * _LANES, _LANES), _LANES)]
            rev = lax.rev(tot, (0,))
            cum = plsc.cumsum(rev)
            cond = (run + cum) >= need
            pop = _scal(plsc.all_reduce_population_count(cond))
            has = pop > 0
            pos = plsc.all_reduce_ffs(cond)
            pos = jnp.where(has, _scal(pos), 0)
            cpos = _at_lane(cum, pos)
            rpos = _at_lane(rev, pos)
            hit = jnp.logical_and(has, jnp.logical_not(found))
            beta = jnp.where(hit, dv * 16 + (15 - pos), beta)
            sfx = run + cpos
            gt_at = jnp.where(hit, sfx - rpos, gt_at)
            sfx_at = jnp.where(hit, sfx, sfx_at)
            found = jnp.logical_or(found, has)
            run = run + _at_lane(cum, 15)
            return run, found, beta, gt_at, sfx_at

        init = (jnp.int32(0), jnp.bool_(False), jnp.int32(0), jnp.int32(0),
                jnp.int32(0))
        _, _, beta, gt_r, _ = lax.fori_loop(0, 16, scan_dv, init)

        prefix = (pfx << jnp.uint32(8)) | jnp.uint32(beta.astype(jnp.uint32))
        need = need - gt_r
        total_gt = total_gt + gt_r

    T_u = prefix
    need_eq = need

    def cbody(i, carry):
        off_gt, off_eq = carry
        uk = load_uk(i)
        kv = keys_v[pl.ds(pl.multiple_of(i * _LANES, _LANES), _LANES)]
        nv = t * SHARD + i * _LANES + li
        m_gt = uk > T_u
        m_eq = uk == T_u
        plsc.store_compressed(gtk_v.at[pl.ds(off_gt, _LANES)], kv, mask=m_gt)
        plsc.store_compressed(gti_v.at[pl.ds(off_gt, _LANES)], nv, mask=m_gt)

        @pl.when(off_eq < M_POINTS)
        def _():
            plsc.store_compressed(eqi_v.at[pl.ds(off_eq, _LANES)], nv,
                                  mask=m_eq)

        off_gt = off_gt + _scal(plsc.all_reduce_population_count(m_gt))
        off_eq = off_eq + _scal(plsc.all_reduce_population_count(m_eq))
        return off_gt, off_eq

    gt_cnt, eq_cnt = lax.fori_loop(0, _NVREG, cbody,
                                   (jnp.int32(0), jnp.int32(0)))

    cnt_v[...] = jnp.where(li == 0, gt_cnt,
                           jnp.where(li == 1, eq_cnt, jnp.int32(0)))
    pltpu.sync_copy(cnt_v, sh_cnt.at[bloc, t])
    plsc.subcore_barrier()
    pltpu.sync_copy(sh_cnt.at[bloc], c8_v)

    gt_base = jnp.int32(0)
    eq_base = jnp.int32(0)
    gt_tot = jnp.int32(0)
    for tt in range(8):
        row = c8_v[tt]
        gcnt = row[0]
        ecnt = row[1]
        before = jnp.int32(tt) < t
        gt_base = gt_base + jnp.where(before, gcnt, 0)
        eq_base = eq_base + jnp.where(before, ecnt, 0)
        gt_tot = gt_tot + gcnt

    wid = c * 16 + s
    obase = batch * M_POINTS

    def _scatter(dst_hbm):
        for j in range(8):
            pltpu.async_copy(valb_v.at[pl.ds(j * 128, 128)],
                             dst_hbm.at[posb_v.at[j]], sem)
        pltpu.make_async_copy(dst_hbm.at[pl.ds(0, M_POINTS)], valb_v,
                              sem).wait()

    for j in range(M_POINTS // _LANES):
        jj = j * _LANES + li
        valid = jj < gt_cnt
        pos = jnp.where(valid, obase + gt_base + jj, _DUMP + wid * 16 + li)
        posb_v[j // 8, pl.ds((j % 8) * _LANES, _LANES)] = pos
        valb_v[pl.ds(j * _LANES, _LANES)] = gtk_v[pl.ds(j * _LANES, _LANES)]
    _scatter(outk_hbm)
    for j in range(M_POINTS // _LANES):
        valb_v[pl.ds(j * _LANES, _LANES)] = gti_v[pl.ds(j * _LANES, _LANES)]
    _scatter(outi_hbm)

    t_signed = plsc.bitcast(
        jnp.zeros((_LANES,), jnp.uint32) + (T_u ^ jnp.uint32(0x80000000)),
        jnp.int32)
    for j in range(M_POINTS // _LANES):
        jj = j * _LANES + li
        erank = eq_base + jj
        valid = jnp.logical_and(jj < eq_cnt, erank < need_eq)
        pos = jnp.where(valid, obase + gt_tot + erank,
                        _DUMP + wid * 16 + li)
        posb_v[j // 8, pl.ds((j % 8) * _LANES, _LANES)] = pos
        valb_v[pl.ds(j * _LANES, _LANES)] = eqi_v[pl.ds(j * _LANES, _LANES)]
    _scatter(outi_hbm)
    for j in range(M_POINTS // _LANES):
        valb_v[pl.ds(j * _LANES, _LANES)] = t_signed
    _scatter(outk_hbm)


def _run_select(keys_flat):
    mesh = plsc.VectorSubcoreMesh(core_axis_name="c", subcore_axis_name="s")
    f = pl.kernel(
        _select_body,
        compiler_params=pltpu.CompilerParams(needs_layout_passes=False),
        out_type=[
            jax.ShapeDtypeStruct((4 * M_POINTS + 512,), jnp.int32),
            jax.ShapeDtypeStruct((4 * M_POINTS + 512,), jnp.int32),
        ],
        mesh=mesh,
        scratch_types=[
            pltpu.VMEM((SHARD,), jnp.int32),
            pltpu.VMEM((4096,), jnp.int32),
            pltpu.VMEM((256,), jnp.int32),
            pltpu.VMEM((8, 256), jnp.int32),
            pltpu.VMEM((16,), jnp.int32),
            pltpu.VMEM((8, 16), jnp.int32),
            pltpu.VMEM((M_POINTS + 16,), jnp.int32),
            pltpu.VMEM((M_POINTS + 16,), jnp.int32),
            pltpu.VMEM((M_POINTS + 16,), jnp.int32),
            pltpu.VMEM((8, 128), jnp.int32),
            pltpu.VMEM((M_POINTS,), jnp.int32),
            pltpu.VMEM_SHARED((4, 2, 8, 256), jnp.int32),
            pltpu.VMEM_SHARED((2, 8, 16), jnp.int32),
            pltpu.SemaphoreType.DMA,
        ],
    )
    return f(keys_flat)


def _rank_body(kj_ref, ki_ref, ci_ref, out_ref):
    kj = kj_ref[0]
    ki = ki_ref[0]
    gt = (kj > ki).astype(jnp.int32)
    ii = jax.lax.broadcasted_iota(jnp.int32, (M_POINTS, M_POINTS), 0)
    jj = jax.lax.broadcasted_iota(jnp.int32, (M_POINTS, M_POINTS), 1)
    eq = jnp.logical_and(kj == ki, jj < ii).astype(jnp.int32)
    rank = jnp.sum(gt + eq, axis=1, keepdims=True)
    onehot = (rank == jj).astype(jnp.float32)
    ci = ci_ref[0].astype(jnp.float32)
    sorted_f = jax.lax.dot_general(
        ci, onehot, (((1,), (0,)), ((), ())),
        preferred_element_type=jnp.float32)
    out_ref[0] = sorted_f.astype(jnp.int32)


def _run_rank(ck, ckT, ci):
    B = ck.shape[0]
    return pl.pallas_call(
        _rank_body,
        grid=(B,),
        in_specs=[
            pl.BlockSpec((1, 1, M_POINTS), lambda bi: (bi, 0, 0)),
            pl.BlockSpec((1, M_POINTS, 1), lambda bi: (bi, 0, 0)),
            pl.BlockSpec((1, 1, M_POINTS), lambda bi: (bi, 0, 0)),
        ],
        out_specs=pl.BlockSpec((1, 1, M_POINTS), lambda bi: (bi, 0, 0)),
        out_shape=jax.ShapeDtypeStruct((B, 1, M_POINTS), jnp.int32),
    )(ck, ckT, ci)


_PTS = 128
_DEBUG_DUMP = False


def _gather_body(pc_hbm, ft_hbm, gr_hbm, si_hbm, fo_hbm, xo_hbm,
                 sidx_v, fidx_v, orow_v, xbuf_v, pci_v, sem):
    c = lax.axis_index("c")
    s = lax.axis_index("s")
    wid = s * 2 + c
    batch = wid // 8
    m0 = (wid % 8) * _PTS

    li = _lane_iota()
    pltpu.sync_copy(si_hbm.at[pl.ds(batch * M_POINTS + m0, _PTS)], sidx_v)

    f_base = batch * C_DIM * N_REAL
    pc_base = batch * N_REAL * 3
    g_base = batch * NPAD

    def grow(p, _):
        nv = sidx_v[pl.ds((p // _LANES) * _LANES, _LANES)]
        n_p = _sum_scal(jnp.where(li == p % _LANES, nv, 0))
        base = f_base + n_p
        for v in range(C_DIM // _LANES):
            fidx_v[pl.ds(v * _LANES, _LANES)] = \
                base + (v * _LANES + li) * N_REAL
        pltpu.async_copy(ft_hbm.at[fidx_v],
                         orow_v.at[pl.ds(p * C_DIM, C_DIM)], sem).wait()
        return 0
    lax.fori_loop(0, _PTS, grow, 0)

    for e in range(3):
        for pv in range(_PTS // _LANES):
            nv = sidx_v[pl.ds(pv * _LANES, _LANES)]
            pci_v[pl.ds(pv * _LANES, _LANES)] = pc_base + nv * 3 + e
        pltpu.async_copy(pc_hbm.at[pci_v],
                         xbuf_v.at[pl.ds(e * _PTS, _PTS)], sem).wait()
    for pv in range(_PTS // _LANES):
        nv = sidx_v[pl.ds(pv * _LANES, _LANES)]
        pci_v[pl.ds(pv * _LANES, _LANES)] = g_base + nv
    pltpu.async_copy(gr_hbm.at[pci_v],
                     xbuf_v.at[pl.ds(3 * _PTS, _PTS)], sem).wait()

    pltpu.sync_copy(orow_v,
                    fo_hbm.at[pl.ds(wid * _PTS * C_DIM, _PTS * C_DIM)])
    pltpu.sync_copy(xbuf_v, xo_hbm.at[pl.ds(wid * 4 * _PTS, 4 * _PTS)])


def _run_gather(pc_flat, ft_flat, gr_flat, si_flat):
    mesh = plsc.VectorSubcoreMesh(core_axis_name="c", subcore_axis_name="s")
    f = pl.kernel(
        _gather_body,
        compiler_params=pltpu.CompilerParams(needs_layout_passes=False),
        out_type=[
            jax.ShapeDtypeStruct((4 * M_POINTS * C_DIM,), jnp.float32),
            jax.ShapeDtypeStruct((32 * 4 * _PTS,), jnp.float32),
        ],
        mesh=mesh,
        scratch_types=[
            pltpu.VMEM((_PTS,), jnp.int32),
            pltpu.VMEM((C_DIM,), jnp.int32),
            pltpu.VMEM((_PTS * C_DIM,), jnp.float32),
            pltpu.VMEM((4 * _PTS,), jnp.float32),
            pltpu.VMEM((_PTS,), jnp.int32),
            pltpu.SemaphoreType.DMA,
        ],
    )
    return f(pc_flat, ft_flat, gr_flat, si_flat)


def kernel(point_clouds, features, W, b):
    B, C, N = features.shape
    keys, grasp = _run_head(features, W, b)
    keys_flat = keys.reshape(B * NPAD)
    ck_flat, ci_flat = _run_select(keys_flat)
    ck = ck_flat[: B * M_POINTS].reshape(B, 1, M_POINTS)
    ckT = ck.reshape(B, M_POINTS, 1)
    ci = ci_flat[: B * M_POINTS].reshape(B, 1, M_POINTS)
    sidx = _run_rank(ck, ckT, ci)
    idx = sidx.reshape(B, M_POINTS)
    xyz = jnp.take_along_axis(point_clouds, idx[:, :, None], axis=1)
    feats = jnp.take_along_axis(
        jnp.transpose(features, (0, 2, 1)), idx[:, :, None], axis=1)
    scores = jnp.take_along_axis(
        grasp.reshape(B, NPAD), idx, axis=1)[:, :, None]
    return jnp.concatenate([xyz, feats, scores], axis=-1)
